# baseline (speedup 1.0000x reference)
import functools
import math

import jax
import jax.numpy as jnp
from jax import lax
from jax.experimental import pallas as pl
from jax.experimental.pallas import tpu as pltpu
from jax.experimental.pallas import tpu_sc as plsc

_D = 64
_SCALE = math.sqrt(_D)
_NW = 32
_BLK = 128
_LANES = 16


def _emb_body(x_hbm, lut2_hbm, out_hbm, idx_v, pidx_v, offs_v, g_v, t_v, sem):
    n_seq = out_hbm.shape[0]
    per_w = _BLK * n_seq
    wid = lax.axis_index("s") * 2 + lax.axis_index("c")

    pltpu.sync_copy(x_hbm.at[pl.ds(wid * per_w, per_w)], idx_v)

    lane = lax.iota(jnp.int32, _LANES)
    lane_row = lane * n_seq

    @pl.loop(0, n_seq)
    def _seq(s):
        for grp in range(_BLK // _LANES):
            flat = lane_row + (grp * _LANES * n_seq + s)
            r = plsc.load_gather(idx_v, [flat])
            pidx_v[pl.ds(grp * _LANES, _LANES)] = lax.shift_right_logical(r, 1)
            offs_v[pl.ds(grp * _LANES, _LANES)] = lax.shift_left(
                jnp.bitwise_and(r, 1), 6
            )

        pltpu.async_copy(lut2_hbm.at[pidx_v], g_v, sem).wait()

        for grp in range(_BLK // _LANES):
            rows = lane + grp * _LANES
            offs = offs_v[pl.ds(grp * _LANES, _LANES)]

            @pl.loop(0, _D)
            def _feat(c):
                v = plsc.load_gather(g_v, [rows, offs + c])
                t_v[c, pl.ds(grp * _LANES, _LANES)] = v * _SCALE

        pltpu.sync_copy(t_v, out_hbm.at[s, :, pl.ds(wid * _BLK, _BLK)])


def kernel(x, lut):
    b, s = x.shape
    vocab, d = lut.shape
    x_flat = x.reshape(b * s)
    lut2 = lut.reshape(vocab // 2, 2 * d)

    mesh = plsc.VectorSubcoreMesh(core_axis_name="c", subcore_axis_name="s")
    run = functools.partial(
        pl.kernel,
        out_type=jax.ShapeDtypeStruct((s, d, b), jnp.float32),
        mesh=mesh,
        scratch_types=[
            pltpu.VMEM((b * s // _NW,), jnp.int32),
            pltpu.VMEM((_BLK,), jnp.int32),
            pltpu.VMEM((_BLK,), jnp.int32),
            pltpu.VMEM((_BLK, 2 * d), jnp.float32),
            pltpu.VMEM((d, _BLK), jnp.float32),
            pltpu.SemaphoreType.DMA,
        ],
        compiler_params=pltpu.CompilerParams(needs_layout_passes=False),
    )(_emb_body)
    out = run(x_flat, lut2)
    return out.transpose(2, 0, 1)
